# baseline (speedup 1.0000x reference)
import numpy as np
import jax
import jax.numpy as jnp
from jax.experimental import pallas as pl
from jax.experimental.pallas import tpu as pltpu

B = 128
NBO = 8
BS = 256
NINP = 1024
GH = 3 * BS
NH = 4
DHID = NBO * BS
TOPK = 4

BF = jnp.bfloat16
F32 = jnp.float32


def _attn_consts():
    seg = np.zeros((NBO * 64, NBO * NH), np.float32)
    for j in range(NBO):
        for h in range(NH):
            seg[j * 64 + h * 16: j * 64 + h * 16 + 16, j * NH + h] = 0.25
    g = np.zeros((NBO * NH, NBO * NH), np.float32)
    for c in range(NBO * NH):
        for c2 in range(NBO * NH):
            if c % NH == c2 % NH:
                g[c, c2] = 1.0
    ebig = np.zeros((NBO * NH, NBO * 64), np.float32)
    for j in range(NBO):
        for h in range(NH):
            ebig[j * NH + h, j * 64 + h * 16: j * 64 + h * 16 + 16] = 1.0
    f = np.zeros((NBO * 64, 64), np.float32)
    for j in range(NBO):
        f[j * 64:(j + 1) * 64, :] = np.eye(64, dtype=np.float32)
    return seg, g, ebig, f


_SEG, _G, _EBIG, _F = _attn_consts()


def _dot(a, b):
    return jax.lax.dot(a, b, preferred_element_type=F32)


def _dott(a, bt):
    return jax.lax.dot_general(a, bt, (((1,), (1,)), ((), ())),
                               preferred_element_type=F32)


def _b(x):
    return x.astype(BF).astype(F32)


def _core(inp_ref, hx_ref, cx_ref, wqt_ref, wk1t_ref, ia_wv_ref,
          mwqt_hbm, mwkt_hbm, mwvt_hbm, wfc_hbm, bfc_ref, wg_hbm, bg_ref,
          wi_hbm, wh_hbm, bi_ref, bh_ref,
          seg_hbm, g_hbm, ebig_hbm, f_hbm,
          hx_out_ref, cx_out_ref, mask_out_ref,
          wi_s, wh_s, mwqt_ref, mwkt_ref, mwvt_ref, wfc_s, wg_s,
          seg_s, g_s, ebig_s, f_s,
          sem_i, sem_h, sem_m):
    small = [(mwqt_hbm, mwqt_ref), (mwkt_hbm, mwkt_ref),
             (mwvt_hbm, mwvt_ref), (wfc_hbm, wfc_s), (wg_hbm, wg_s),
             (seg_hbm, seg_s), (g_hbm, g_s), (ebig_hbm, ebig_s),
             (f_hbm, f_s)]
    for i, (src_r, dst_r) in enumerate(small):
        pltpu.make_async_copy(src_r, dst_r, sem_m.at[i]).start()
    for j in range(NBO):
        pltpu.make_async_copy(wi_hbm.at[j], wi_s.at[j], sem_i.at[j]).start()
        pltpu.make_async_copy(wh_hbm.at[j], wh_s.at[j], sem_h.at[j]).start()

    inp = inp_ref[...]
    hx = hx_ref[...]

    k1 = _dott(inp, wk1t_ref[0])
    v1 = _dot(inp, ia_wv_ref[0])

    ljs = []
    for j in range(NBO):
        hbj = hx[:, j * BS:(j + 1) * BS]
        qj = _dott(hbj, wqt_ref[j])
        ljs.append(jnp.sum(_b(qj) * _b(k1), axis=1, keepdims=True) * 0.125)
    logits = jnp.concatenate(ljs, axis=1)

    col = jax.lax.broadcasted_iota(jnp.int32, (B, NBO), 1)
    masks, p1s = [], []
    for j in range(NBO):
        lj = ljs[j]
        below = (logits < lj) | ((logits == lj) & (col < j))
        cnt = jnp.sum(below.astype(F32), axis=1, keepdims=True)
        masks.append((cnt >= TOPK).astype(F32))
        p1s.append(jax.nn.sigmoid(lj))

    hns, qs, ks, vs = [], [], [], []
    for j in range(NBO):
        pltpu.make_async_copy(wi_hbm.at[j], wi_s.at[j], sem_i.at[j]).wait()
        pltpu.make_async_copy(wh_hbm.at[j], wh_s.at[j], sem_h.at[j]).wait()
        hbj = hx[:, j * BS:(j + 1) * BS]
        gi = p1s[j] * _dot(v1, wi_s[j]) + bi_ref[j:j + 1, :]
        gh = _dot(hbj, wh_s[j]) + bh_ref[j:j + 1, :]
        r = jax.nn.sigmoid(gi[:, :BS] + gh[:, :BS])
        z = jax.nn.sigmoid(gi[:, BS:2 * BS] + gh[:, BS:2 * BS])
        n = jnp.tanh(gi[:, 2 * BS:] + r * gh[:, 2 * BS:])
        hn = (1.0 - z) * n + z * hbj
        hns.append(hn)
        if j == 0:
            for i, (src_r, dst_r) in enumerate(small):
                pltpu.make_async_copy(src_r, dst_r, sem_m.at[i]).wait()
        qs.append(_dott(hn, mwqt_ref[j]))
        ks.append(_dott(hn, mwkt_ref[j]))
        vs.append(_dott(hn, mwvt_ref[j]))

    kcat = jnp.concatenate(ks, axis=1)
    vcat = jnp.concatenate(vs, axis=1)
    seg = seg_s[...]
    gmat = g_s[...]
    ebig = ebig_s[...]
    fmat = f_s[...]
    wfc = wfc_s[...]
    wg = wg_s[...]
    bfc = bfc_ref[...]
    bg = bg_ref[...]
    hfin = []
    for i in range(NBO):
        qt = jnp.concatenate([qs[i]] * NBO, axis=1)
        s = _dot(qt * kcat, seg)
        e = jnp.exp(s)
        pn = e / _dot(e, gmat)
        w = _dot(pn, ebig)
        out = _dot(w * vcat, fmat)
        o = _dot(out, wfc) + bfc
        a = _dot(out, wg) + bg
        hfin.append(hns[i] + jax.nn.sigmoid(a) * jnp.tanh(o))

    cx = cx_ref[...]
    for j in range(NBO):
        m = masks[j]
        sl = slice(j * BS, (j + 1) * BS)
        hx_out_ref[:, sl] = m * hfin[j] + (1.0 - m) * hx[:, sl]
        cx_out_ref[:, sl] = m * hns[j] + (1.0 - m) * cx[:, sl]
        mask_out_ref[:, sl] = jnp.broadcast_to(m, (B, BS))


def kernel(inp, hx, cx, ia_wq, ia_wk, ia_wv, mha_wq, mha_wk, mha_wv,
           mha_wfc, mha_bfc, mha_wg, mha_bg, gru_wi, gru_wh, gru_bi,
           gru_bh, step):
    vmem = pl.BlockSpec(memory_space=pltpu.VMEM)
    out_shape = [jax.ShapeDtypeStruct((B, DHID), F32) for _ in range(3)]
    hx_out, cx_out, mask = pl.pallas_call(
        _core,
        grid=(1,),
        in_specs=[
            vmem, vmem, vmem,
            vmem,
            pl.BlockSpec((1, 64, NINP), lambda i: (1, 0, 0)),
            pl.BlockSpec((1, NINP, NINP), lambda i: (1, 0, 0)),
            pl.BlockSpec(memory_space=pl.ANY),
            pl.BlockSpec(memory_space=pl.ANY),
            pl.BlockSpec(memory_space=pl.ANY),
            pl.BlockSpec(memory_space=pl.ANY),
            vmem,
            pl.BlockSpec(memory_space=pl.ANY),
            vmem,
            pl.BlockSpec(memory_space=pl.ANY),
            pl.BlockSpec(memory_space=pl.ANY),
            vmem, vmem,
            pl.BlockSpec(memory_space=pl.ANY),
            pl.BlockSpec(memory_space=pl.ANY),
            pl.BlockSpec(memory_space=pl.ANY),
            pl.BlockSpec(memory_space=pl.ANY),
        ],
        out_specs=[vmem] * 3,
        out_shape=out_shape,
        scratch_shapes=[
            pltpu.VMEM((NBO, NINP, GH), F32),
            pltpu.VMEM((NBO, BS, GH), F32),
            pltpu.VMEM((NBO, 64, BS), F32),
            pltpu.VMEM((NBO, 64, BS), F32),
            pltpu.VMEM((NBO, 64, BS), F32),
            pltpu.VMEM((64, BS), F32),
            pltpu.VMEM((64, BS), F32),
            pltpu.VMEM((NBO * 64, NBO * NH), F32),
            pltpu.VMEM((NBO * NH, NBO * NH), F32),
            pltpu.VMEM((NBO * NH, NBO * 64), F32),
            pltpu.VMEM((NBO * 64, 64), F32),
            pltpu.SemaphoreType.DMA((NBO,)),
            pltpu.SemaphoreType.DMA((NBO,)),
            pltpu.SemaphoreType.DMA((9,)),
        ],
    )(inp, hx, cx,
      ia_wq.transpose(0, 2, 1), ia_wk.transpose(0, 2, 1), ia_wv,
      mha_wq.transpose(0, 2, 1), mha_wk.transpose(0, 2, 1),
      mha_wv.transpose(0, 2, 1),
      mha_wfc, mha_bfc.reshape(1, BS), mha_wg, mha_bg.reshape(1, BS),
      gru_wi, gru_wh, gru_bi, gru_bh,
      jnp.asarray(_SEG), jnp.asarray(_G), jnp.asarray(_EBIG),
      jnp.asarray(_F))
    return hx_out, cx_out, mask


# speedup vs baseline: 1.2233x; 1.2233x over previous
import numpy as np
import jax
import jax.numpy as jnp
from jax.experimental import pallas as pl
from jax.experimental.pallas import tpu as pltpu

B = 128
NBO = 8
BS = 256
NINP = 1024
GH = 3 * BS
NH = 4
DHID = NBO * BS
TOPK = 4

BF = jnp.bfloat16
F32 = jnp.float32


def _attn_consts():
    seg = np.zeros((NBO * 64, NBO * NH), np.float32)
    for j in range(NBO):
        for h in range(NH):
            seg[j * 64 + h * 16: j * 64 + h * 16 + 16, j * NH + h] = 0.25
    g = np.zeros((NBO * NH, NBO * NH), np.float32)
    for c in range(NBO * NH):
        for c2 in range(NBO * NH):
            if c % NH == c2 % NH:
                g[c, c2] = 1.0
    ebig = np.zeros((NBO * NH, NBO * 64), np.float32)
    for j in range(NBO):
        for h in range(NH):
            ebig[j * NH + h, j * 64 + h * 16: j * 64 + h * 16 + 16] = 1.0
    f = np.zeros((NBO * 64, 64), np.float32)
    for j in range(NBO):
        f[j * 64:(j + 1) * 64, :] = np.eye(64, dtype=np.float32)
    return seg, g, ebig, f


_SEG, _G, _EBIG, _F = _attn_consts()


def _dot(a, b):
    return jax.lax.dot(a, b, preferred_element_type=F32)


def _dott(a, bt):
    return jax.lax.dot_general(a, bt, (((1,), (1,)), ((), ())),
                               preferred_element_type=F32)


def _b(x):
    return x.astype(BF).astype(F32)


def _core(inp_ref, hx_ref, cx_ref, wqt_ref, wk1t_ref, ia_wv_ref,
          mwqt_ref, mwkt_ref, mwvt_ref, wfc_ref, bfc_ref, wg_ref, bg_ref,
          wi_hbm, wh_hbm, bi_ref, bh_ref,
          seg_ref, g_ref, ebig_ref, f_ref,
          hx_out_ref, cx_out_ref, mask_out_ref,
          wi_s, wh_s, sem_i, sem_h):
    for j in range(NBO):
        pltpu.make_async_copy(wi_hbm.at[j], wi_s.at[j], sem_i.at[j]).start()
        pltpu.make_async_copy(wh_hbm.at[j], wh_s.at[j], sem_h.at[j]).start()

    inp = inp_ref[...]
    hx = hx_ref[...]

    k1 = _dott(inp, wk1t_ref[0])
    v1 = _dot(inp, ia_wv_ref[0])

    ljs = []
    for j in range(NBO):
        hbj = hx[:, j * BS:(j + 1) * BS]
        qj = _dott(hbj, wqt_ref[j])
        ljs.append(jnp.sum(_b(qj) * _b(k1), axis=1, keepdims=True) * 0.125)
    logits = jnp.concatenate(ljs, axis=1)

    col = jax.lax.broadcasted_iota(jnp.int32, (B, NBO), 1)
    masks, p1s = [], []
    for j in range(NBO):
        lj = ljs[j]
        below = (logits < lj) | ((logits == lj) & (col < j))
        cnt = jnp.sum(below.astype(F32), axis=1, keepdims=True)
        masks.append((cnt >= TOPK).astype(F32))
        p1s.append(jax.nn.sigmoid(lj))

    hns, qs, ks, vs = [], [], [], []
    for j in range(NBO):
        pltpu.make_async_copy(wi_hbm.at[j], wi_s.at[j], sem_i.at[j]).wait()
        pltpu.make_async_copy(wh_hbm.at[j], wh_s.at[j], sem_h.at[j]).wait()
        hbj = hx[:, j * BS:(j + 1) * BS]
        gi = p1s[j] * _dot(v1, wi_s[j]) + bi_ref[j:j + 1, :]
        gh = _dot(hbj, wh_s[j]) + bh_ref[j:j + 1, :]
        r = jax.nn.sigmoid(gi[:, :BS] + gh[:, :BS])
        z = jax.nn.sigmoid(gi[:, BS:2 * BS] + gh[:, BS:2 * BS])
        n = jnp.tanh(gi[:, 2 * BS:] + r * gh[:, 2 * BS:])
        hn = (1.0 - z) * n + z * hbj
        hns.append(hn)
        qs.append(_dott(hn, mwqt_ref[j]))
        ks.append(_dott(hn, mwkt_ref[j]))
        vs.append(_dott(hn, mwvt_ref[j]))

    kcat = jnp.concatenate(ks, axis=1)
    vcat = jnp.concatenate(vs, axis=1)
    seg = seg_ref[...]
    gmat = g_ref[...]
    ebig = ebig_ref[...]
    fmat = f_ref[...]
    wfc = wfc_ref[...]
    wg = wg_ref[...]
    bfc = bfc_ref[...]
    bg = bg_ref[...]
    hfin = []
    for i in range(NBO):
        qt = jnp.concatenate([qs[i]] * NBO, axis=1)
        s = _dot(qt * kcat, seg)
        e = jnp.exp(s)
        pn = e / _dot(e, gmat)
        w = _dot(pn, ebig)
        out = _dot(w * vcat, fmat)
        o = _dot(out, wfc) + bfc
        a = _dot(out, wg) + bg
        hfin.append(hns[i] + jax.nn.sigmoid(a) * jnp.tanh(o))

    cx = cx_ref[...]
    for j in range(NBO):
        m = masks[j]
        sl = slice(j * BS, (j + 1) * BS)
        hx_out_ref[:, sl] = m * hfin[j] + (1.0 - m) * hx[:, sl]
        cx_out_ref[:, sl] = m * hns[j] + (1.0 - m) * cx[:, sl]
        mask_out_ref[:, sl] = jnp.broadcast_to(m, (B, BS))


def kernel(inp, hx, cx, ia_wq, ia_wk, ia_wv, mha_wq, mha_wk, mha_wv,
           mha_wfc, mha_bfc, mha_wg, mha_bg, gru_wi, gru_wh, gru_bi,
           gru_bh, step):
    vmem = pl.BlockSpec(memory_space=pltpu.VMEM)
    out_shape = [jax.ShapeDtypeStruct((B, DHID), F32) for _ in range(3)]
    hx_out, cx_out, mask = pl.pallas_call(
        _core,
        grid=(1,),
        in_specs=[
            vmem, vmem, vmem,
            vmem,
            pl.BlockSpec((1, 64, NINP), lambda i: (1, 0, 0)),
            pl.BlockSpec((1, NINP, NINP), lambda i: (1, 0, 0)),
            vmem, vmem, vmem,
            vmem, vmem, vmem, vmem,
            pl.BlockSpec(memory_space=pl.ANY),
            pl.BlockSpec(memory_space=pl.ANY),
            vmem, vmem,
            vmem, vmem, vmem, vmem,
        ],
        out_specs=[vmem] * 3,
        out_shape=out_shape,
        scratch_shapes=[
            pltpu.VMEM((NBO, NINP, GH), F32),
            pltpu.VMEM((NBO, BS, GH), F32),
            pltpu.SemaphoreType.DMA((NBO,)),
            pltpu.SemaphoreType.DMA((NBO,)),
        ],
    )(inp, hx, cx,
      ia_wq.transpose(0, 2, 1), ia_wk.transpose(0, 2, 1), ia_wv,
      mha_wq.transpose(0, 2, 1), mha_wk.transpose(0, 2, 1),
      mha_wv.transpose(0, 2, 1),
      mha_wfc, mha_bfc.reshape(1, BS), mha_wg, mha_bg.reshape(1, BS),
      gru_wi, gru_wh, gru_bi, gru_bh,
      jnp.asarray(_SEG), jnp.asarray(_G), jnp.asarray(_EBIG),
      jnp.asarray(_F))
    return hx_out, cx_out, mask
